# Initial kernel scaffold; baseline (speedup 1.0000x reference)
import jax
import jax.numpy as jnp
from jax.experimental import pallas as pl


def kernel(x, edge_index, batch, W1, b1, W2, b2, W3, b3, LW1, Lb1, LW2, Lb2, LW3, Lb3):
    raise NotImplementedError("write your pallas kernel here")


# speedup vs baseline: 16.7018x; 16.7018x over previous
import functools

import jax
import jax.numpy as jnp
from jax import lax
from jax.experimental import pallas as pl
from jax.experimental.pallas import tpu as pltpu
from jax.experimental.pallas import tpu_sc as plsc

N = 10000
E = 320000
F = 128
NG = 64
NCLS = 10

NC = 2
NS = 16
NW = NC * NS
EPW = E // NW
CH = 125
NCH = EPW // CH
RPT = 624
RPT_LAST = N - 15 * RPT

@functools.lru_cache(maxsize=None)
def _sc_kernels():
    mesh = plsc.VectorSubcoreMesh(core_axis_name="c", subcore_axis_name="s",
                                  num_cores=NC, num_subcores=NS)

    @functools.partial(
        pl.kernel,
        out_type=jax.ShapeDtypeStruct((NC * N,), jnp.float32),
        mesh=mesh,
        scratch_types=[
            pltpu.VMEM((NCH, CH), jnp.int32),
            pltpu.VMEM((CH,), jnp.float32),
            pltpu.VMEM((N,), jnp.float32),
            pltpu.VMEM_SHARED((N,), jnp.float32),
        ],
    )
    def sc_deg(dst_hbm, ones_hbm, zn_hbm, out_hbm, dst_v, ones_v, stage_v, acc):
        c = lax.axis_index("c")
        s = lax.axis_index("s")
        wid = s * NC + c
        pltpu.sync_copy(dst_hbm.at[wid], dst_v)
        pltpu.sync_copy(ones_hbm, ones_v)

        @pl.when(s == 0)
        def _():
            pltpu.sync_copy(zn_hbm, stage_v)
            pltpu.sync_copy(stage_v, acc)

        plsc.subcore_barrier()

        @pl.loop(0, NCH)
        def _(j):
            pltpu.sync_copy(ones_v, acc.at[dst_v.at[j]], add=True)

        plsc.subcore_barrier()

        @pl.when(s == 0)
        def _():
            pltpu.sync_copy(acc, stage_v)
            pltpu.sync_copy(stage_v, out_hbm.at[pl.ds(c * N, N)])

    @functools.partial(
        pl.kernel,
        out_type=jax.ShapeDtypeStruct((NC, N, F), jnp.float32),
        mesh=mesh,
        scratch_types=[
            pltpu.VMEM((NCH, CH), jnp.int32),
            pltpu.VMEM((NCH, CH), jnp.int32),
            pltpu.VMEM((CH, F), jnp.float32),
            pltpu.VMEM_SHARED((N, F), jnp.float32),
            pltpu.SemaphoreType.DMA,
        ],
    )
    def sc_edge(u_hbm, src_hbm, dst_hbm, zrow_hbm, out_hbm,
                src_v, dst_v, rows_v, acc, sem):
        c = lax.axis_index("c")
        s = lax.axis_index("s")
        wid = s * NC + c
        pltpu.sync_copy(src_hbm.at[wid], src_v)
        pltpu.sync_copy(dst_hbm.at[wid], dst_v)

        @pl.when(s < NS - 1)
        def _():
            pltpu.sync_copy(zrow_hbm.at[pl.ds(0, RPT)],
                            acc.at[pl.ds(s * RPT, RPT)])

        @pl.when(s == NS - 1)
        def _():
            pltpu.sync_copy(zrow_hbm, acc.at[pl.ds(15 * RPT, RPT_LAST)])

        plsc.subcore_barrier()

        @pl.loop(0, NCH)
        def _(j):
            pltpu.async_copy(u_hbm.at[src_v.at[j]], rows_v, sem).wait()
            pltpu.sync_copy(rows_v, acc.at[dst_v.at[j]], add=True)

        plsc.subcore_barrier()

        @pl.when(s < NS - 1)
        def _():
            pltpu.sync_copy(acc.at[pl.ds(s * RPT, RPT)],
                            out_hbm.at[c, pl.ds(s * RPT, RPT)])

        @pl.when(s == NS - 1)
        def _():
            pltpu.sync_copy(acc.at[pl.ds(15 * RPT, RPT_LAST)],
                            out_hbm.at[c, pl.ds(15 * RPT, RPT_LAST)])

    return sc_deg, sc_edge


R = 2000
G = N // R


def _tc_pre(d0, d1, x, W):
    def body(d0_ref, d1_ref, x_ref, w_ref, u_ref, dis_ref):
        dis = lax.rsqrt(d0_ref[...] + d1_ref[...] + 1.0)
        dis_ref[...] = dis
        u_ref[...] = dis * jnp.dot(x_ref[...], w_ref[...],
                                   preferred_element_type=jnp.float32)

    return pl.pallas_call(
        body,
        grid=(G,),
        in_specs=[
            pl.BlockSpec((R, 1), lambda i: (i, 0)),
            pl.BlockSpec((R, 1), lambda i: (i, 0)),
            pl.BlockSpec((R, F), lambda i: (i, 0)),
            pl.BlockSpec((F, F), lambda i: (0, 0)),
        ],
        out_specs=[
            pl.BlockSpec((R, F), lambda i: (i, 0)),
            pl.BlockSpec((R, 1), lambda i: (i, 0)),
        ],
        out_shape=[
            jax.ShapeDtypeStruct((N, F), jnp.float32),
            jax.ShapeDtypeStruct((N, 1), jnp.float32),
        ],
    )(d0, d1, x, W)


def _tc_mid(p0, p1, u, dis, b, W):
    def body(p0_ref, p1_ref, u_ref, dis_ref, b_ref, w_ref, un_ref):
        dis = dis_ref[...]
        h = jnp.maximum(
            dis * (p0_ref[...] + p1_ref[...] + u_ref[...]) + b_ref[...], 0.0)
        un_ref[...] = dis * jnp.dot(h, w_ref[...],
                                    preferred_element_type=jnp.float32)

    return pl.pallas_call(
        body,
        grid=(G,),
        in_specs=[
            pl.BlockSpec((R, F), lambda i: (i, 0)),
            pl.BlockSpec((R, F), lambda i: (i, 0)),
            pl.BlockSpec((R, F), lambda i: (i, 0)),
            pl.BlockSpec((R, 1), lambda i: (i, 0)),
            pl.BlockSpec((1, F), lambda i: (0, 0)),
            pl.BlockSpec((F, F), lambda i: (0, 0)),
        ],
        out_specs=pl.BlockSpec((R, F), lambda i: (i, 0)),
        out_shape=jax.ShapeDtypeStruct((N, F), jnp.float32),
    )(p0, p1, u, dis, b, W)


def _tc_pool(p0, p1, u, dis, b, bb):
    def body(p0_ref, p1_ref, u_ref, dis_ref, b_ref, bb_ref,
             gmax_ref, gsum_ref, cnt_ref):
        i = pl.program_id(0)
        h = jnp.maximum(
            dis_ref[...] * (p0_ref[...] + p1_ref[...] + u_ref[...])
            + b_ref[...], 0.0)
        bbv = bb_ref[...]
        onehot = (bbv == lax.broadcasted_iota(jnp.int32, (1, NG), 1)
                  ).astype(jnp.float32)
        gs = lax.dot_general(onehot, h, (((0,), (0,)), ((), ())),
                             preferred_element_type=jnp.float32)
        cn = jnp.sum(onehot, axis=0).reshape(NG, 1)

        @pl.when(i == 0)
        def _():
            gmax_ref[...] = jnp.full((NG, F), -jnp.inf, jnp.float32)
            gsum_ref[...] = jnp.zeros((NG, F), jnp.float32)
            cnt_ref[...] = jnp.zeros((NG, 1), jnp.float32)

        gsum_ref[...] += gs
        cnt_ref[...] += cn
        rows = [jnp.max(jnp.where(bbv == g, h, -jnp.inf), axis=0,
                        keepdims=True) for g in range(NG)]
        gmax_ref[...] = jnp.maximum(gmax_ref[...],
                                    jnp.concatenate(rows, axis=0))

    return pl.pallas_call(
        body,
        grid=(G,),
        in_specs=[
            pl.BlockSpec((R, F), lambda i: (i, 0)),
            pl.BlockSpec((R, F), lambda i: (i, 0)),
            pl.BlockSpec((R, F), lambda i: (i, 0)),
            pl.BlockSpec((R, 1), lambda i: (i, 0)),
            pl.BlockSpec((1, F), lambda i: (0, 0)),
            pl.BlockSpec((R, 1), lambda i: (i, 0)),
        ],
        out_specs=[
            pl.BlockSpec((NG, F), lambda i: (0, 0)),
            pl.BlockSpec((NG, F), lambda i: (0, 0)),
            pl.BlockSpec((NG, 1), lambda i: (0, 0)),
        ],
        out_shape=[
            jax.ShapeDtypeStruct((NG, F), jnp.float32),
            jax.ShapeDtypeStruct((NG, F), jnp.float32),
            jax.ShapeDtypeStruct((NG, 1), jnp.float32),
        ],
    )(p0, p1, u, dis, b, bb)


def _tc_head(gmax, gsum, cnt, la, lb, b1, w2, b2, w3, b3):
    def body(gmax_ref, gsum_ref, cnt_ref, la_ref, lb_ref, b1_ref,
             w2_ref, b2_ref, w3_ref, b3_ref, out_ref):
        gmean = gsum_ref[...] / jnp.maximum(cnt_ref[...], 1.0)
        z = jnp.maximum(
            jnp.dot(gmax_ref[...], la_ref[...],
                    preferred_element_type=jnp.float32)
            + jnp.dot(gmean, lb_ref[...],
                      preferred_element_type=jnp.float32)
            + b1_ref[...], 0.0)
        z = jnp.maximum(jnp.dot(z, w2_ref[...],
                                preferred_element_type=jnp.float32)
                        + b2_ref[...], 0.0)
        lg = jnp.dot(z, w3_ref[...],
                     preferred_element_type=jnp.float32) + b3_ref[...]
        m = jnp.max(lg, axis=-1, keepdims=True)
        e = jnp.exp(lg - m)
        out_ref[...] = (lg - m) - jnp.log(jnp.sum(e, axis=-1, keepdims=True))

    return pl.pallas_call(
        body,
        out_shape=jax.ShapeDtypeStruct((NG, NCLS), jnp.float32),
    )(gmax, gsum, cnt, la, lb, b1, w2, b2, w3, b3)


def kernel(x, edge_index, batch, W1, b1, W2, b2, W3, b3,
           LW1, Lb1, LW2, Lb2, LW3, Lb3):
    src = edge_index[0].reshape(NW, NCH, CH)
    dst = edge_index[1].reshape(NW, NCH, CH)
    ones_ch = jnp.ones((CH,), jnp.float32)
    zn = jnp.zeros((N,), jnp.float32)
    zrow = jnp.zeros((RPT_LAST, F), jnp.float32)

    sc_deg, sc_edge = _sc_kernels()
    degp = sc_deg(dst, ones_ch, zn).reshape(NC, N)
    d0 = degp[0].reshape(N, 1)
    d1 = degp[1].reshape(N, 1)

    u1, dis = _tc_pre(d0, d1, x, W1)
    p = sc_edge(u1, src, dst, zrow)
    u2 = _tc_mid(p[0], p[1], u1, dis, b1.reshape(1, F), W2)
    p = sc_edge(u2, src, dst, zrow)
    u3 = _tc_mid(p[0], p[1], u2, dis, b2.reshape(1, F), W3)
    p = sc_edge(u3, src, dst, zrow)
    gmax, gsum, cnt = _tc_pool(p[0], p[1], u3, dis, b3.reshape(1, F),
                               batch.reshape(N, 1))
    return _tc_head(gmax, gsum, cnt, LW1[:F], LW1[F:],
                    Lb1.reshape(1, -1), LW2, Lb2.reshape(1, -1),
                    LW3, Lb3.reshape(1, -1))


# speedup vs baseline: 21.3907x; 1.2807x over previous
import functools

import jax
import jax.numpy as jnp
from jax import lax
from jax.experimental import pallas as pl
from jax.experimental.pallas import tpu as pltpu
from jax.experimental.pallas import tpu_sc as plsc

N = 10000
E = 320000
F = 128
NG = 64
NCLS = 10

NC = 2
NS = 16
NW = NC * NS
EPW = E // NW
CH = 80
NCH = EPW // CH
RPT = 624
RPT_LAST = N - 15 * RPT

@functools.lru_cache(maxsize=None)
def _sc_kernels():
    mesh = plsc.VectorSubcoreMesh(core_axis_name="c", subcore_axis_name="s",
                                  num_cores=NC, num_subcores=NS)

    @functools.partial(
        pl.kernel,
        out_type=jax.ShapeDtypeStruct((NC * N,), jnp.float32),
        mesh=mesh,
        scratch_types=[
            pltpu.VMEM((NCH, CH), jnp.int32),
            pltpu.VMEM((CH,), jnp.float32),
            pltpu.VMEM((N,), jnp.float32),
            pltpu.VMEM_SHARED((N,), jnp.float32),
        ],
    )
    def sc_deg(dst_hbm, ones_hbm, zn_hbm, out_hbm, dst_v, ones_v, stage_v, acc):
        c = lax.axis_index("c")
        s = lax.axis_index("s")
        wid = s * NC + c
        pltpu.sync_copy(dst_hbm.at[wid], dst_v)
        pltpu.sync_copy(ones_hbm, ones_v)

        @pl.when(s == 0)
        def _():
            pltpu.sync_copy(zn_hbm, stage_v)
            pltpu.sync_copy(stage_v, acc)

        plsc.subcore_barrier()

        @pl.loop(0, NCH)
        def _(j):
            pltpu.sync_copy(ones_v, acc.at[dst_v.at[j]], add=True)

        plsc.subcore_barrier()

        @pl.when(s == 0)
        def _():
            pltpu.sync_copy(acc, stage_v)
            pltpu.sync_copy(stage_v, out_hbm.at[pl.ds(c * N, N)])

    @functools.partial(
        pl.kernel,
        out_type=jax.ShapeDtypeStruct((NC, N, F), jnp.float32),
        mesh=mesh,
        scratch_types=[
            pltpu.VMEM((EPW,), jnp.int32),
            pltpu.VMEM((NCH, CH), jnp.int32),
            pltpu.VMEM((CH, F), jnp.float32),
            pltpu.VMEM((CH, F), jnp.float32),
            pltpu.VMEM_SHARED((N, F), jnp.float32),
            pltpu.SemaphoreType.DMA,
            pltpu.SemaphoreType.DMA,
        ],
    )
    def sc_edge(u_hbm, src_hbm, dst_hbm, zrow_hbm, out_hbm,
                src_v, dst_v, rows_a, rows_b, acc, sem_a, sem_b):
        c = lax.axis_index("c")
        s = lax.axis_index("s")
        wid = s * NC + c
        pltpu.sync_copy(src_hbm.at[pl.ds(wid * EPW, EPW)], src_v)
        pltpu.sync_copy(dst_hbm.at[wid], dst_v)

        @pl.when(s < NS - 1)
        def _():
            pltpu.sync_copy(zrow_hbm.at[pl.ds(0, RPT)],
                            acc.at[pl.ds(s * RPT, RPT)])

        @pl.when(s == NS - 1)
        def _():
            pltpu.sync_copy(zrow_hbm, acc.at[pl.ds(15 * RPT, RPT_LAST)])

        plsc.subcore_barrier()

        def sidx(j):
            return src_v.at[pl.ds(j * CH, CH)]

        pltpu.async_copy(u_hbm.at[sidx(0)], rows_a, sem_a)

        @pl.loop(0, NCH - 1, step=2)
        def _(j):
            pltpu.make_async_copy(u_hbm.at[sidx(j)], rows_a, sem_a).wait()
            pltpu.async_copy(u_hbm.at[sidx(j + 1)], rows_b, sem_b)
            pltpu.sync_copy(rows_a, acc.at[dst_v.at[j]], add=True)
            pltpu.async_copy(u_hbm.at[sidx(j + 2)], rows_a, sem_a)
            pltpu.make_async_copy(u_hbm.at[sidx(j + 1)], rows_b,
                                  sem_b).wait()
            pltpu.sync_copy(rows_b, acc.at[dst_v.at[j + 1]], add=True)

        pltpu.make_async_copy(u_hbm.at[sidx(NCH - 1)], rows_a, sem_a).wait()
        pltpu.sync_copy(rows_a, acc.at[dst_v.at[NCH - 1]], add=True)

        plsc.subcore_barrier()

        @pl.when(s < NS - 1)
        def _():
            pltpu.sync_copy(acc.at[pl.ds(s * RPT, RPT)],
                            out_hbm.at[c, pl.ds(s * RPT, RPT)])

        @pl.when(s == NS - 1)
        def _():
            pltpu.sync_copy(acc.at[pl.ds(15 * RPT, RPT_LAST)],
                            out_hbm.at[c, pl.ds(15 * RPT, RPT_LAST)])

    return sc_deg, sc_edge


R = 2000
G = N // R


def _tc_pre(d0, d1, x, W):
    def body(d0_ref, d1_ref, x_ref, w_ref, u_ref, dis_ref):
        dis = lax.rsqrt(d0_ref[...] + d1_ref[...] + 1.0)
        dis_ref[...] = dis
        u_ref[...] = dis * jnp.dot(x_ref[...], w_ref[...],
                                   preferred_element_type=jnp.float32)

    return pl.pallas_call(
        body,
        grid=(G,),
        in_specs=[
            pl.BlockSpec((R, 1), lambda i: (i, 0)),
            pl.BlockSpec((R, 1), lambda i: (i, 0)),
            pl.BlockSpec((R, F), lambda i: (i, 0)),
            pl.BlockSpec((F, F), lambda i: (0, 0)),
        ],
        out_specs=[
            pl.BlockSpec((R, F), lambda i: (i, 0)),
            pl.BlockSpec((R, 1), lambda i: (i, 0)),
        ],
        out_shape=[
            jax.ShapeDtypeStruct((N, F), jnp.float32),
            jax.ShapeDtypeStruct((N, 1), jnp.float32),
        ],
    )(d0, d1, x, W)


def _tc_mid(p0, p1, u, dis, b, W):
    def body(p0_ref, p1_ref, u_ref, dis_ref, b_ref, w_ref, un_ref):
        dis = dis_ref[...]
        h = jnp.maximum(
            dis * (p0_ref[...] + p1_ref[...] + u_ref[...]) + b_ref[...], 0.0)
        un_ref[...] = dis * jnp.dot(h, w_ref[...],
                                    preferred_element_type=jnp.float32)

    return pl.pallas_call(
        body,
        grid=(G,),
        in_specs=[
            pl.BlockSpec((R, F), lambda i: (i, 0)),
            pl.BlockSpec((R, F), lambda i: (i, 0)),
            pl.BlockSpec((R, F), lambda i: (i, 0)),
            pl.BlockSpec((R, 1), lambda i: (i, 0)),
            pl.BlockSpec((1, F), lambda i: (0, 0)),
            pl.BlockSpec((F, F), lambda i: (0, 0)),
        ],
        out_specs=pl.BlockSpec((R, F), lambda i: (i, 0)),
        out_shape=jax.ShapeDtypeStruct((N, F), jnp.float32),
    )(p0, p1, u, dis, b, W)


def _tc_pool(p0, p1, u, dis, b, bb):
    def body(p0_ref, p1_ref, u_ref, dis_ref, b_ref, bb_ref,
             gmax_ref, gsum_ref, cnt_ref):
        i = pl.program_id(0)
        h = jnp.maximum(
            dis_ref[...] * (p0_ref[...] + p1_ref[...] + u_ref[...])
            + b_ref[...], 0.0)
        bbv = bb_ref[...]
        onehot = (bbv == lax.broadcasted_iota(jnp.int32, (1, NG), 1)
                  ).astype(jnp.float32)
        gs = lax.dot_general(onehot, h, (((0,), (0,)), ((), ())),
                             preferred_element_type=jnp.float32)
        cn = jnp.sum(onehot, axis=0).reshape(NG, 1)

        @pl.when(i == 0)
        def _():
            gmax_ref[...] = jnp.full((NG, F), -jnp.inf, jnp.float32)
            gsum_ref[...] = jnp.zeros((NG, F), jnp.float32)
            cnt_ref[...] = jnp.zeros((NG, 1), jnp.float32)

        gsum_ref[...] += gs
        cnt_ref[...] += cn
        rows = [jnp.max(jnp.where(bbv == g, h, -jnp.inf), axis=0,
                        keepdims=True) for g in range(NG)]
        gmax_ref[...] = jnp.maximum(gmax_ref[...],
                                    jnp.concatenate(rows, axis=0))

    return pl.pallas_call(
        body,
        grid=(G,),
        in_specs=[
            pl.BlockSpec((R, F), lambda i: (i, 0)),
            pl.BlockSpec((R, F), lambda i: (i, 0)),
            pl.BlockSpec((R, F), lambda i: (i, 0)),
            pl.BlockSpec((R, 1), lambda i: (i, 0)),
            pl.BlockSpec((1, F), lambda i: (0, 0)),
            pl.BlockSpec((R, 1), lambda i: (i, 0)),
        ],
        out_specs=[
            pl.BlockSpec((NG, F), lambda i: (0, 0)),
            pl.BlockSpec((NG, F), lambda i: (0, 0)),
            pl.BlockSpec((NG, 1), lambda i: (0, 0)),
        ],
        out_shape=[
            jax.ShapeDtypeStruct((NG, F), jnp.float32),
            jax.ShapeDtypeStruct((NG, F), jnp.float32),
            jax.ShapeDtypeStruct((NG, 1), jnp.float32),
        ],
    )(p0, p1, u, dis, b, bb)


def _tc_head(gmax, gsum, cnt, la, lb, b1, w2, b2, w3, b3):
    def body(gmax_ref, gsum_ref, cnt_ref, la_ref, lb_ref, b1_ref,
             w2_ref, b2_ref, w3_ref, b3_ref, out_ref):
        gmean = gsum_ref[...] / jnp.maximum(cnt_ref[...], 1.0)
        z = jnp.maximum(
            jnp.dot(gmax_ref[...], la_ref[...],
                    preferred_element_type=jnp.float32)
            + jnp.dot(gmean, lb_ref[...],
                      preferred_element_type=jnp.float32)
            + b1_ref[...], 0.0)
        z = jnp.maximum(jnp.dot(z, w2_ref[...],
                                preferred_element_type=jnp.float32)
                        + b2_ref[...], 0.0)
        lg = jnp.dot(z, w3_ref[...],
                     preferred_element_type=jnp.float32) + b3_ref[...]
        m = jnp.max(lg, axis=-1, keepdims=True)
        e = jnp.exp(lg - m)
        out_ref[...] = (lg - m) - jnp.log(jnp.sum(e, axis=-1, keepdims=True))

    return pl.pallas_call(
        body,
        out_shape=jax.ShapeDtypeStruct((NG, NCLS), jnp.float32),
    )(gmax, gsum, cnt, la, lb, b1, w2, b2, w3, b3)


def kernel(x, edge_index, batch, W1, b1, W2, b2, W3, b3,
           LW1, Lb1, LW2, Lb2, LW3, Lb3):
    src = edge_index[0]
    dst = edge_index[1].reshape(NW, NCH, CH)
    ones_ch = jnp.ones((CH,), jnp.float32)
    zn = jnp.zeros((N,), jnp.float32)
    zrow = jnp.zeros((RPT_LAST, F), jnp.float32)

    sc_deg, sc_edge = _sc_kernels()
    degp = sc_deg(dst, ones_ch, zn).reshape(NC, N)
    d0 = degp[0].reshape(N, 1)
    d1 = degp[1].reshape(N, 1)

    u1, dis = _tc_pre(d0, d1, x, W1)
    p = sc_edge(u1, src, dst, zrow)
    u2 = _tc_mid(p[0], p[1], u1, dis, b1.reshape(1, F), W2)
    p = sc_edge(u2, src, dst, zrow)
    u3 = _tc_mid(p[0], p[1], u2, dis, b2.reshape(1, F), W3)
    p = sc_edge(u3, src, dst, zrow)
    gmax, gsum, cnt = _tc_pool(p[0], p[1], u3, dis, b3.reshape(1, F),
                               batch.reshape(N, 1))
    return _tc_head(gmax, gsum, cnt, LW1[:F], LW1[F:],
                    Lb1.reshape(1, -1), LW2, Lb2.reshape(1, -1),
                    LW3, Lb3.reshape(1, -1))


# speedup vs baseline: 22.1833x; 1.0371x over previous
import functools

import jax
import jax.numpy as jnp
from jax import lax
from jax.experimental import pallas as pl
from jax.experimental.pallas import tpu as pltpu
from jax.experimental.pallas import tpu_sc as plsc

N = 10000
E = 320000
F = 128
NG = 64
NCLS = 10

NC = 2
NS = 16
NW = NC * NS
EPW = E // NW
CH = 80
NCH = EPW // CH
RPT = 624
RPT_LAST = N - 15 * RPT

@functools.lru_cache(maxsize=None)
def _sc_kernels():
    mesh = plsc.VectorSubcoreMesh(core_axis_name="c", subcore_axis_name="s",
                                  num_cores=NC, num_subcores=NS)

    @functools.partial(
        pl.kernel,
        out_type=jax.ShapeDtypeStruct((NC * N,), jnp.float32),
        mesh=mesh,
        scratch_types=[
            pltpu.VMEM((NCH, CH), jnp.int32),
            pltpu.VMEM((CH,), jnp.float32),
            pltpu.VMEM((N,), jnp.float32),
            pltpu.VMEM_SHARED((N,), jnp.float32),
        ],
    )
    def sc_deg(dst_hbm, ones_hbm, zn_hbm, out_hbm, dst_v, ones_v, stage_v, acc):
        c = lax.axis_index("c")
        s = lax.axis_index("s")
        wid = s * NC + c
        pltpu.sync_copy(dst_hbm.at[wid], dst_v)
        pltpu.sync_copy(ones_hbm, ones_v)

        @pl.when(s == 0)
        def _():
            pltpu.sync_copy(zn_hbm, stage_v)
            pltpu.sync_copy(stage_v, acc)

        plsc.subcore_barrier()

        @pl.loop(0, NCH)
        def _(j):
            pltpu.sync_copy(ones_v, acc.at[dst_v.at[j]], add=True)

        plsc.subcore_barrier()

        @pl.when(s == 0)
        def _():
            pltpu.sync_copy(acc, stage_v)
            pltpu.sync_copy(stage_v, out_hbm.at[pl.ds(c * N, N)])

    @functools.partial(
        pl.kernel,
        out_type=jax.ShapeDtypeStruct((NC, N, F), jnp.float32),
        mesh=mesh,
        scratch_types=[
            pltpu.VMEM((EPW,), jnp.int32),
            pltpu.VMEM((NCH, CH), jnp.int32),
            pltpu.VMEM((CH, F), jnp.float32),
            pltpu.VMEM((CH, F), jnp.float32),
            pltpu.VMEM_SHARED((N, F), jnp.float32),
            pltpu.SemaphoreType.DMA,
            pltpu.SemaphoreType.DMA,
        ],
    )
    def sc_edge(u_hbm, src_hbm, dst_hbm, zrow_hbm, out_hbm,
                src_v, dst_v, rows_a, rows_b, acc, sem_a, sem_b):
        c = lax.axis_index("c")
        s = lax.axis_index("s")
        wid = s * NC + c
        pltpu.sync_copy(src_hbm.at[pl.ds(wid * EPW, EPW)], src_v)
        pltpu.sync_copy(dst_hbm.at[wid], dst_v)

        @pl.when(s < NS - 1)
        def _():
            pltpu.sync_copy(zrow_hbm.at[pl.ds(0, RPT)],
                            acc.at[pl.ds(s * RPT, RPT)])

        @pl.when(s == NS - 1)
        def _():
            pltpu.sync_copy(zrow_hbm, acc.at[pl.ds(15 * RPT, RPT_LAST)])

        plsc.subcore_barrier()

        def sidx(j):
            return src_v.at[pl.ds(j * CH, CH)]

        pltpu.async_copy(u_hbm.at[sidx(0)], rows_a, sem_a)

        @pl.loop(0, NCH - 1, step=2)
        def _(j):
            pltpu.make_async_copy(u_hbm.at[sidx(j)], rows_a, sem_a).wait()
            pltpu.async_copy(u_hbm.at[sidx(j + 1)], rows_b, sem_b)
            pltpu.sync_copy(rows_a, acc.at[dst_v.at[j]], add=True)
            pltpu.async_copy(u_hbm.at[sidx(j + 2)], rows_a, sem_a)
            pltpu.make_async_copy(u_hbm.at[sidx(j + 1)], rows_b,
                                  sem_b).wait()
            pltpu.sync_copy(rows_b, acc.at[dst_v.at[j + 1]], add=True)

        pltpu.make_async_copy(u_hbm.at[sidx(NCH - 1)], rows_a, sem_a).wait()
        pltpu.sync_copy(rows_a, acc.at[dst_v.at[NCH - 1]], add=True)

        plsc.subcore_barrier()

        @pl.when(s < NS - 1)
        def _():
            pltpu.sync_copy(acc.at[pl.ds(s * RPT, RPT)],
                            out_hbm.at[c, pl.ds(s * RPT, RPT)])

        @pl.when(s == NS - 1)
        def _():
            pltpu.sync_copy(acc.at[pl.ds(15 * RPT, RPT_LAST)],
                            out_hbm.at[c, pl.ds(15 * RPT, RPT_LAST)])

    return sc_deg, sc_edge


R = 2000
G = N // R


def _tc_pre(d, x, W):
    def body(d_ref, x_ref, w_ref, u_ref, dis_ref):
        dis = lax.rsqrt(d_ref[0] + d_ref[1] + 1.0)
        dis_ref[...] = dis
        u_ref[...] = dis * jnp.dot(x_ref[...], w_ref[...],
                                   preferred_element_type=jnp.float32)

    return pl.pallas_call(
        body,
        grid=(G,),
        in_specs=[
            pl.BlockSpec((NC, R, 1), lambda i: (0, i, 0)),
            pl.BlockSpec((R, F), lambda i: (i, 0)),
            pl.BlockSpec((F, F), lambda i: (0, 0)),
        ],
        out_specs=[
            pl.BlockSpec((R, F), lambda i: (i, 0)),
            pl.BlockSpec((R, 1), lambda i: (i, 0)),
        ],
        out_shape=[
            jax.ShapeDtypeStruct((N, F), jnp.float32),
            jax.ShapeDtypeStruct((N, 1), jnp.float32),
        ],
    )(d, x, W)


def _tc_mid(p, u, dis, b, W):
    def body(p_ref, u_ref, dis_ref, b_ref, w_ref, un_ref):
        dis = dis_ref[...]
        h = jnp.maximum(
            dis * (p_ref[0] + p_ref[1] + u_ref[...]) + b_ref[...], 0.0)
        un_ref[...] = dis * jnp.dot(h, w_ref[...],
                                    preferred_element_type=jnp.float32)

    return pl.pallas_call(
        body,
        grid=(G,),
        in_specs=[
            pl.BlockSpec((NC, R, F), lambda i: (0, i, 0)),
            pl.BlockSpec((R, F), lambda i: (i, 0)),
            pl.BlockSpec((R, 1), lambda i: (i, 0)),
            pl.BlockSpec((1, F), lambda i: (0, 0)),
            pl.BlockSpec((F, F), lambda i: (0, 0)),
        ],
        out_specs=pl.BlockSpec((R, F), lambda i: (i, 0)),
        out_shape=jax.ShapeDtypeStruct((N, F), jnp.float32),
    )(p, u, dis, b, W)


def _tc_pool(p, u, dis, b, bb):
    def body(p_ref, u_ref, dis_ref, b_ref, bb_ref,
             gmax_ref, gsum_ref, cnt_ref):
        i = pl.program_id(0)
        h = jnp.maximum(
            dis_ref[...] * (p_ref[0] + p_ref[1] + u_ref[...])
            + b_ref[...], 0.0)
        bbv = bb_ref[...]
        onehot = (bbv == lax.broadcasted_iota(jnp.int32, (1, NG), 1)
                  ).astype(jnp.float32)
        gs = lax.dot_general(onehot, h, (((0,), (0,)), ((), ())),
                             preferred_element_type=jnp.float32)
        cn = jnp.sum(onehot, axis=0).reshape(NG, 1)

        @pl.when(i == 0)
        def _():
            gmax_ref[...] = jnp.full((NG, F), -jnp.inf, jnp.float32)
            gsum_ref[...] = jnp.zeros((NG, F), jnp.float32)
            cnt_ref[...] = jnp.zeros((NG, 1), jnp.float32)

        gsum_ref[...] += gs
        cnt_ref[...] += cn
        rows = [jnp.max(jnp.where(bbv == g, h, -jnp.inf), axis=0,
                        keepdims=True) for g in range(NG)]
        gmax_ref[...] = jnp.maximum(gmax_ref[...],
                                    jnp.concatenate(rows, axis=0))

    return pl.pallas_call(
        body,
        grid=(G,),
        in_specs=[
            pl.BlockSpec((NC, R, F), lambda i: (0, i, 0)),
            pl.BlockSpec((R, F), lambda i: (i, 0)),
            pl.BlockSpec((R, 1), lambda i: (i, 0)),
            pl.BlockSpec((1, F), lambda i: (0, 0)),
            pl.BlockSpec((R, 1), lambda i: (i, 0)),
        ],
        out_specs=[
            pl.BlockSpec((NG, F), lambda i: (0, 0)),
            pl.BlockSpec((NG, F), lambda i: (0, 0)),
            pl.BlockSpec((NG, 1), lambda i: (0, 0)),
        ],
        out_shape=[
            jax.ShapeDtypeStruct((NG, F), jnp.float32),
            jax.ShapeDtypeStruct((NG, F), jnp.float32),
            jax.ShapeDtypeStruct((NG, 1), jnp.float32),
        ],
    )(p, u, dis, b, bb)


def _tc_head(gmax, gsum, cnt, la, lb, b1, w2, b2, w3, b3):
    def body(gmax_ref, gsum_ref, cnt_ref, la_ref, lb_ref, b1_ref,
             w2_ref, b2_ref, w3_ref, b3_ref, out_ref):
        gmean = gsum_ref[...] / jnp.maximum(cnt_ref[...], 1.0)
        z = jnp.maximum(
            jnp.dot(gmax_ref[...], la_ref[...],
                    preferred_element_type=jnp.float32)
            + jnp.dot(gmean, lb_ref[...],
                      preferred_element_type=jnp.float32)
            + b1_ref[...], 0.0)
        z = jnp.maximum(jnp.dot(z, w2_ref[...],
                                preferred_element_type=jnp.float32)
                        + b2_ref[...], 0.0)
        lg = jnp.dot(z, w3_ref[...],
                     preferred_element_type=jnp.float32) + b3_ref[...]
        m = jnp.max(lg, axis=-1, keepdims=True)
        e = jnp.exp(lg - m)
        out_ref[...] = (lg - m) - jnp.log(jnp.sum(e, axis=-1, keepdims=True))

    return pl.pallas_call(
        body,
        out_shape=jax.ShapeDtypeStruct((NG, NCLS), jnp.float32),
    )(gmax, gsum, cnt, la, lb, b1, w2, b2, w3, b3)


def kernel(x, edge_index, batch, W1, b1, W2, b2, W3, b3,
           LW1, Lb1, LW2, Lb2, LW3, Lb3):
    src = edge_index[0]
    dst = edge_index[1].reshape(NW, NCH, CH)
    ones_ch = jnp.ones((CH,), jnp.float32)
    zn = jnp.zeros((N,), jnp.float32)
    zrow = jnp.zeros((RPT_LAST, F), jnp.float32)

    sc_deg, sc_edge = _sc_kernels()
    degp = sc_deg(dst, ones_ch, zn).reshape(NC, N, 1)

    u1, dis = _tc_pre(degp, x, W1)
    p = sc_edge(u1, src, dst, zrow)
    u2 = _tc_mid(p, u1, dis, b1.reshape(1, F), W2)
    p = sc_edge(u2, src, dst, zrow)
    u3 = _tc_mid(p, u2, dis, b2.reshape(1, F), W3)
    p = sc_edge(u3, src, dst, zrow)
    gmax, gsum, cnt = _tc_pool(p, u3, dis, b3.reshape(1, F),
                               batch.reshape(N, 1))
    return _tc_head(gmax, gsum, cnt, LW1[:F], LW1[F:],
                    Lb1.reshape(1, -1), LW2, Lb2.reshape(1, -1),
                    LW3, Lb3.reshape(1, -1))
